# baseline (device time: 63640 ns/iter reference)
import jax
import jax.numpy as jnp
from jax import lax
from jax.experimental import pallas as pl
from jax.experimental.pallas import tpu as pltpu

N_DEV = 8
B = 2
S_PER = 128
S_GLOB = N_DEV * S_PER
HQ = 4
DH = 64
LOCAL_WINDOW = 128
GLOBAL_PREFIX = 32


def kernel(x, Wq, K_ext, V_ext, Wo):
    d_model = x.shape[-1]
    d_qk = Wq.shape[-1]

    def body(x_ref, wq_ref, k_ref, v_ref, wo_ref, out_ref,
             kvh_ref, send_sems, recv_sems):
        my = lax.axis_index("i")
        left = lax.rem(my - 1 + N_DEV, N_DEV)
        right = lax.rem(my + 1, N_DEV)

        barrier_sem = pltpu.get_barrier_semaphore()
        for nbr in (left, right):
            pl.semaphore_signal(
                barrier_sem, inc=1,
                device_id=(nbr,), device_id_type=pl.DeviceIdType.MESH,
            )
        pl.semaphore_wait(barrier_sem, 2)

        for j in range(HQ):
            kvh_ref[j, :, pl.ds(my * S_PER, S_PER), :] = (
                k_ref[:, :, j, :].astype(jnp.bfloat16))
            kvh_ref[HQ + j, :, pl.ds(my * S_PER, S_PER), :] = (
                v_ref[:, :, j, :].astype(jnp.bfloat16))

        for h in range(N_DEV - 1):
            origin = lax.rem(my - h + N_DEV, N_DEV)
            rdma = pltpu.make_async_remote_copy(
                src_ref=kvh_ref.at[:, :, pl.ds(origin * S_PER, S_PER), :],
                dst_ref=kvh_ref.at[:, :, pl.ds(origin * S_PER, S_PER), :],
                send_sem=send_sems.at[h],
                recv_sem=recv_sems.at[h],
                device_id=(right,),
                device_id_type=pl.DeviceIdType.MESH,
            )
            rdma.start()
            rdma.wait()

        xb = x_ref[:].astype(jnp.bfloat16)
        wq = wq_ref[:].astype(jnp.bfloat16)
        q = lax.dot_general(
            xb, wq, (((2,), (0,)), ((), ())),
            preferred_element_type=jnp.float32,
        )

        qi = lax.broadcasted_iota(jnp.int32, (S_PER, S_GLOB), 0) + my * S_PER
        ki = lax.broadcasted_iota(jnp.int32, (S_PER, S_GLOB), 1)
        mask = ((jnp.abs(qi - ki) <= LOCAL_WINDOW)
                | (ki < GLOBAL_PREFIX) | (qi < GLOBAL_PREFIX))

        acc = jnp.zeros((B, S_PER, d_model), dtype=jnp.float32)
        for h in range(HQ):
            qh = q[:, :, h * DH:(h + 1) * DH].astype(jnp.bfloat16)
            kh = kvh_ref[h]
            vh = kvh_ref[HQ + h]
            s = lax.dot_general(
                qh, kh, (((2,), (2,)), ((0,), (0,))),
                preferred_element_type=jnp.float32,
            ) * 0.125
            s = jnp.where(mask[None, :, :], s, -1e9)
            s_max = jnp.max(s, axis=-1, keepdims=True)
            w = jnp.exp(s - s_max)
            w = w / jnp.sum(w, axis=-1, keepdims=True)
            ctx = lax.dot_general(
                w.astype(jnp.bfloat16), vh, (((2,), (1,)), ((0,), (0,))),
                preferred_element_type=jnp.float32,
            )
            woh = wo_ref[h * DH:(h + 1) * DH, :].astype(jnp.bfloat16)
            acc = acc + lax.dot_general(
                ctx.astype(jnp.bfloat16), woh, (((2,), (0,)), ((), ())),
                preferred_element_type=jnp.float32,
            )
        out_ref[:] = acc

    out_shape = jax.ShapeDtypeStruct((B, S_PER, d_model), jnp.float32)
    return pl.pallas_call(
        body,
        out_shape=out_shape,
        in_specs=[pl.BlockSpec(memory_space=pltpu.VMEM)] * 5,
        out_specs=pl.BlockSpec(memory_space=pltpu.VMEM),
        scratch_shapes=[
            pltpu.VMEM((2 * HQ, B, S_GLOB, DH), jnp.bfloat16),
            pltpu.SemaphoreType.DMA((N_DEV - 1,)),
            pltpu.SemaphoreType.DMA((N_DEV - 1,)),
        ],
        compiler_params=pltpu.CompilerParams(collective_id=0),
    )(x, Wq, K_ext, V_ext, Wo)


# device time: 38790 ns/iter; 1.6406x vs baseline; 1.6406x over previous
import jax
import jax.numpy as jnp
from jax import lax
from jax.experimental import pallas as pl
from jax.experimental.pallas import tpu as pltpu

N_DEV = 8
B = 2
S_PER = 128
S_GLOB = N_DEV * S_PER
HQ = 4
DH = 64
LOCAL_WINDOW = 128
GLOBAL_PREFIX = 32
R_HOPS = 4
L_HOPS = 3


def kernel(x, Wq, K_ext, V_ext, Wo):
    d_model = x.shape[-1]

    def body(x_ref, wq_ref, k_ref, v_ref, wo_ref, out_ref,
             kvh_ref, r_send, r_recv, l_send, l_recv):
        my = lax.axis_index("i")
        left = lax.rem(my - 1 + N_DEV, N_DEV)
        right = lax.rem(my + 1, N_DEV)

        barrier_sem = pltpu.get_barrier_semaphore()
        for nbr in (left, right):
            pl.semaphore_signal(
                barrier_sem, inc=1,
                device_id=(nbr,), device_id_type=pl.DeviceIdType.MESH,
            )
        pl.semaphore_wait(barrier_sem, 2)

        for j in range(HQ):
            kvh_ref[j, :, pl.ds(my * S_PER, S_PER), :] = (
                k_ref[:, :, j, :].astype(jnp.bfloat16))
            kvh_ref[HQ + j, :, pl.ds(my * S_PER, S_PER), :] = (
                v_ref[:, :, j, :].astype(jnp.bfloat16))

        def block(origin):
            return kvh_ref.at[:, :, pl.ds(origin * S_PER, S_PER), :]

        def make_rdma(origin, sems_s, sems_r, h, target):
            return pltpu.make_async_remote_copy(
                src_ref=block(origin), dst_ref=block(origin),
                send_sem=sems_s.at[h], recv_sem=sems_r.at[h],
                device_id=(target,), device_id_type=pl.DeviceIdType.MESH,
            )

        def org(k):
            return lax.rem(my + k + N_DEV, N_DEV)

        r_rdma = [make_rdma(org(-h), r_send, r_recv, h, right)
                  for h in range(R_HOPS)]
        l_rdma = [make_rdma(org(+h), l_send, l_recv, h, left)
                  for h in range(L_HOPS)]
        r_rdma[0].start()
        l_rdma[0].start()

        xb = x_ref[:].astype(jnp.bfloat16)
        wq = wq_ref[:].astype(jnp.bfloat16)
        q = lax.dot_general(
            xb, wq, (((2,), (0,)), ((), ())),
            preferred_element_type=jnp.float32,
        )
        qh = [q[:, :, h * DH:(h + 1) * DH].astype(jnp.bfloat16)
              for h in range(HQ)]

        qi_loc = lax.broadcasted_iota(jnp.int32, (S_PER, S_PER), 0)
        ki_loc = lax.broadcasted_iota(jnp.int32, (S_PER, S_PER), 1)
        qi_glob = qi_loc + my * S_PER

        num = [jnp.zeros((B, S_PER, DH), jnp.float32) for _ in range(HQ)]
        den = [jnp.zeros((B, S_PER, 1), jnp.float32) for _ in range(HQ)]

        def accumulate(origin):
            ko = origin * S_PER
            ki_glob = ki_loc + ko
            mask = ((jnp.abs(qi_glob - ki_glob) <= LOCAL_WINDOW)
                    | (ki_glob < GLOBAL_PREFIX) | (qi_glob < GLOBAL_PREFIX))
            maskf = mask.astype(jnp.float32)[None, :, :]
            for h in range(HQ):
                kb = kvh_ref[h, :, pl.ds(ko, S_PER), :]
                vb = kvh_ref[HQ + h, :, pl.ds(ko, S_PER), :]
                s = lax.dot_general(
                    qh[h], kb, (((2,), (2,)), ((0,), (0,))),
                    preferred_element_type=jnp.float32,
                ) * 0.125
                e = jnp.exp(s) * maskf
                num[h] = num[h] + lax.dot_general(
                    e.astype(jnp.bfloat16), vb, (((2,), (1,)), ((0,), (0,))),
                    preferred_element_type=jnp.float32,
                )
                den[h] = den[h] + jnp.sum(e, axis=-1, keepdims=True)

        accumulate(org(0))

        for h in range(R_HOPS):
            r_rdma[h].wait_recv()
            if h + 1 < R_HOPS:
                r_rdma[h + 1].start()
            if h < L_HOPS:
                l_rdma[h].wait_recv()
                if h + 1 < L_HOPS:
                    l_rdma[h + 1].start()
            accumulate(org(-1 - h))
            if h < L_HOPS:
                accumulate(org(+1 + h))

        for r in r_rdma:
            r.wait_send()
        for r in l_rdma:
            r.wait_send()

        acc = jnp.zeros((B, S_PER, d_model), dtype=jnp.float32)
        for h in range(HQ):
            ctx = (num[h] / den[h]).astype(jnp.bfloat16)
            woh = wo_ref[h * DH:(h + 1) * DH, :].astype(jnp.bfloat16)
            acc = acc + lax.dot_general(
                ctx, woh, (((2,), (0,)), ((), ())),
                preferred_element_type=jnp.float32,
            )
        out_ref[:] = acc

    out_shape = jax.ShapeDtypeStruct((B, S_PER, d_model), jnp.float32)
    return pl.pallas_call(
        body,
        out_shape=out_shape,
        in_specs=[pl.BlockSpec(memory_space=pltpu.VMEM)] * 5,
        out_specs=pl.BlockSpec(memory_space=pltpu.VMEM),
        scratch_shapes=[
            pltpu.VMEM((2 * HQ, B, S_GLOB, DH), jnp.bfloat16),
            pltpu.SemaphoreType.DMA((R_HOPS,)),
            pltpu.SemaphoreType.DMA((R_HOPS,)),
            pltpu.SemaphoreType.DMA((L_HOPS,)),
            pltpu.SemaphoreType.DMA((L_HOPS,)),
        ],
        compiler_params=pltpu.CompilerParams(collective_id=0),
    )(x, Wq, K_ext, V_ext, Wo)
